# baseline (speedup 1.0000x reference)
import functools

import jax
import jax.numpy as jnp
from jax import lax
from jax.experimental import pallas as pl
from jax.experimental.pallas import tpu as pltpu
from jax.experimental.pallas import tpu_sc as plsc

DP = 8


_BIG = 1e30


def _nn_body(n_total, bj, nj, real_ref, pred_ref, idx_ref, vmin_ref, hi_ref,
             idxf_ref, rows_ref, predT_ref):
    b = pl.program_id(0)
    p = pl.program_id(1)
    jj = pl.program_id(2)

    @pl.when((b == 0) & (p == 0) & (jj == 0))
    def _init_rows():
        ri = lax.broadcasted_iota(jnp.int32, (bj, n_total), 0)
        rows_ref[...] = lax.bitcast_convert_type(ri | 0x3F800000, jnp.float32)

    @pl.when((p == 0) & (jj == 0))
    def _transpose_pred():
        predT_ref[...] = lax.transpose(pred_ref[0], (1, 0))

    def _sq():
        r = real_ref[0]
        pT = predT_ref[...]
        r_sq = (r[:, 0:1] * r[:, 0:1] + r[:, 1:2] * r[:, 1:2]
                + r[:, 2:3] * r[:, 2:3])
        p_sq = (pT[0:1, :] * pT[0:1, :] + pT[1:2, :] * pT[1:2, :]
                + pT[2:3, :] * pT[2:3, :])
        cross2 = lax.dot_general(
            r + r, pT, dimension_numbers=(((1,), (0,)), ((), ())),
            preferred_element_type=jnp.float32)
        return (p_sq + r_sq) - cross2

    @pl.when(p == 0)
    def _pass_min():
        m = jnp.min(_sq(), axis=0, keepdims=True)

        @pl.when(jj == 0)
        def _():
            vmin_ref[...] = m

        @pl.when(jj != 0)
        def _():
            vmin_ref[...] = jnp.minimum(vmin_ref[...], m)

    @pl.when(p == 1)
    def _pass_idx():
        @pl.when(jj == 0)
        def _():
            vmin = vmin_ref[...]
            dmin = jnp.sqrt(jnp.maximum(vmin, 0.0))
            c = lax.bitcast_convert_type(dmin * dmin, jnp.int32) + 3
            for _ in range(6):
                cf = lax.bitcast_convert_type(c, jnp.float32)
                c = c - (jnp.sqrt(cf) > dmin).astype(jnp.int32)
            hi = lax.bitcast_convert_type(c, jnp.float32)
            hi_ref[...] = jnp.where(dmin > 0.0, hi, 0.0)
            idxf_ref[...] = jnp.full((1, n_total), 1 << 30, jnp.int32)

        acc = _sq() <= hi_ref[...]
        enc = jnp.where(acc, rows_ref[...], jnp.float32(3.0))
        cand = jnp.min(enc, axis=0, keepdims=True)
        dec = lax.bitcast_convert_type(cand, jnp.int32) & 0x7FFFFF
        merged = jnp.minimum(idxf_ref[...], dec + (b * n_total + jj * bj))
        idxf_ref[...] = merged

        @pl.when(jj == nj - 1)
        def _():
            idx_ref[0] = merged


def _nearest_indices(real_pad, predT_pad, bj):
    B, N, _ = real_pad.shape
    nj = N // bj
    return pl.pallas_call(
        functools.partial(_nn_body, N, bj, nj),
        grid=(B, 2, nj),
        in_specs=[
            pl.BlockSpec((1, bj, 3), lambda b, p, j: (b, j, 0)),
            pl.BlockSpec((1, N, 3), lambda b, p, j: (b, 0, 0)),
        ],
        out_specs=pl.BlockSpec((1, 1, N), lambda b, p, j: (b, 0, 0)),
        out_shape=jax.ShapeDtypeStruct((B, 1, N), jnp.int32),
        scratch_shapes=[
            pltpu.VMEM((1, N), jnp.float32),
            pltpu.VMEM((1, N), jnp.float32),
            pltpu.VMEM((1, N), jnp.int32),
            pltpu.VMEM((bj, N), jnp.float32),
            pltpu.VMEM((3, N), jnp.float32),
        ],
        compiler_params=pltpu.CompilerParams(
            dimension_semantics=("parallel", "arbitrary", "arbitrary")),
    )(real_pad, predT_pad)


def _make_sc_gather(T, G, chunk):
    NC, NS = 2, 16
    nw = NC * NS
    rows_per_w = T // nw
    nchunk = rows_per_w // chunk
    mesh = plsc.VectorSubcoreMesh(core_axis_name="c", subcore_axis_name="s")

    @functools.partial(
        pl.kernel,
        mesh=mesh,
        out_type=jax.ShapeDtypeStruct((T, G), jnp.float32),
        scratch_types=[
            pltpu.VMEM((nchunk, chunk), jnp.int32),
            pltpu.VMEM((chunk, G), jnp.float32),
            pltpu.VMEM((chunk, G), jnp.float32),
            pltpu.VMEM((chunk, G), jnp.float32),
            pltpu.SemaphoreType.DMA,
            pltpu.SemaphoreType.DMA,
            pltpu.SemaphoreType.DMA,
            pltpu.SemaphoreType.DMA,
            pltpu.SemaphoreType.DMA,
            pltpu.SemaphoreType.DMA,
        ],
    )
    def gather_k(idx_hbm, table_hbm, out_hbm, idx_v, rows_a, rows_b, rows_c,
                 sg_a, sg_b, sg_c, sw_a, sw_b, sw_c):
        wid = lax.axis_index("s") * NC + lax.axis_index("c")
        base = wid * rows_per_w
        pltpu.sync_copy(idx_hbm.at[wid], idx_v)
        bufs = (rows_a, rows_b, rows_c)
        sg = (sg_a, sg_b, sg_c)
        sw = (sw_a, sw_b, sw_c)

        def gather(ci):
            return pltpu.async_copy(
                table_hbm.at[idx_v.at[ci]], bufs[ci % 3], sg[ci % 3])

        gcp = [gather(0), gather(1)]
        wcp = []
        for ci in range(nchunk):
            gcp[ci].wait()
            wcp.append(pltpu.async_copy(
                bufs[ci % 3], out_hbm.at[pl.ds(base + ci * chunk, chunk)],
                sw[ci % 3]))
            if ci + 2 < nchunk:
                if ci >= 1:
                    wcp[ci - 1].wait()
                gcp.append(gather(ci + 2))
        for j in range(nchunk - 3, nchunk):
            wcp[j].wait()

    return gather_k


def kernel(predicted_positions, real_positions, real_expressions):
    B, N, D = predicted_positions.shape
    G = real_expressions.shape[-1]

    idx = _nearest_indices(real_positions, predicted_positions, bj=1024)

    table = real_expressions.reshape(B * N, G)
    chunk = 64
    nw = 32
    idx3 = idx.reshape(nw, (B * N) // (nw * chunk), chunk)
    gather_fn = _make_sc_gather(B * N, G, chunk=chunk)
    out = gather_fn(idx3, table)
    return out.reshape(B, N, G)


# speedup vs baseline: 1.0661x; 1.0661x over previous
import functools

import jax
import jax.numpy as jnp
from jax import lax
from jax.experimental import pallas as pl
from jax.experimental.pallas import tpu as pltpu
from jax.experimental.pallas import tpu_sc as plsc

DP = 8


_BIG = 1e30


def _nn_body(n_total, bj, nj, real_ref, predT_ref, idx_ref, vmin_ref, hi_ref,
             idxf_ref, rows_ref):
    b = pl.program_id(0)
    p = pl.program_id(1)
    jj = pl.program_id(2)

    @pl.when((b == 0) & (p == 0) & (jj == 0))
    def _init_rows():
        ri = lax.broadcasted_iota(jnp.int32, (bj, n_total), 0)
        rows_ref[...] = lax.bitcast_convert_type(ri | 0x3F800000, jnp.float32)

    def _sq():
        r = real_ref[0]
        pT = predT_ref[0]
        r_sq = (r[:, 0:1] * r[:, 0:1] + r[:, 1:2] * r[:, 1:2]
                + r[:, 2:3] * r[:, 2:3])
        p_sq = (pT[0:1, :] * pT[0:1, :] + pT[1:2, :] * pT[1:2, :]
                + pT[2:3, :] * pT[2:3, :])
        cross2 = lax.dot_general(
            r + r, pT, dimension_numbers=(((1,), (0,)), ((), ())),
            preferred_element_type=jnp.float32)
        return (p_sq + r_sq) - cross2

    @pl.when(p == 0)
    def _pass_min():
        m = jnp.min(_sq(), axis=0, keepdims=True)

        @pl.when(jj == 0)
        def _():
            vmin_ref[...] = m

        @pl.when(jj != 0)
        def _():
            vmin_ref[...] = jnp.minimum(vmin_ref[...], m)

    @pl.when(p == 1)
    def _pass_idx():
        @pl.when(jj == 0)
        def _():
            vmin = vmin_ref[...]
            dmin = jnp.sqrt(jnp.maximum(vmin, 0.0))
            c = lax.bitcast_convert_type(dmin * dmin, jnp.int32) + 3
            for _ in range(6):
                cf = lax.bitcast_convert_type(c, jnp.float32)
                c = c - (jnp.sqrt(cf) > dmin).astype(jnp.int32)
            hi = lax.bitcast_convert_type(c, jnp.float32)
            hi_ref[...] = jnp.where(dmin > 0.0, hi, 0.0)
            idxf_ref[...] = jnp.full((1, n_total), 1 << 30, jnp.int32)

        acc = _sq() <= hi_ref[...]
        enc = jnp.where(acc, rows_ref[...], jnp.float32(3.0))
        cand = jnp.min(enc, axis=0, keepdims=True)
        dec = lax.bitcast_convert_type(cand, jnp.int32) & 0x7FFFFF
        merged = jnp.minimum(idxf_ref[...], dec + (b * n_total + jj * bj))
        idxf_ref[...] = merged

        @pl.when(jj == nj - 1)
        def _():
            idx_ref[0] = merged


def _nearest_indices(real_pad, predT_pad, bj):
    B, N, _ = real_pad.shape
    nj = N // bj
    return pl.pallas_call(
        functools.partial(_nn_body, N, bj, nj),
        grid=(B, 2, nj),
        in_specs=[
            pl.BlockSpec((1, bj, 3), lambda b, p, j: (b, j, 0)),
            pl.BlockSpec((1, 3, N), lambda b, p, j: (b, 0, 0)),
        ],
        out_specs=pl.BlockSpec((1, 1, N), lambda b, p, j: (b, 0, 0)),
        out_shape=jax.ShapeDtypeStruct((B, 1, N), jnp.int32),
        scratch_shapes=[
            pltpu.VMEM((1, N), jnp.float32),
            pltpu.VMEM((1, N), jnp.float32),
            pltpu.VMEM((1, N), jnp.int32),
            pltpu.VMEM((bj, N), jnp.float32),
        ],
        compiler_params=pltpu.CompilerParams(
            dimension_semantics=("parallel", "arbitrary", "arbitrary")),
    )(real_pad, predT_pad)


def _make_sc_gather(T, G, chunk):
    NC, NS = 2, 16
    nw = NC * NS
    rows_per_w = T // nw
    nchunk = rows_per_w // chunk
    mesh = plsc.VectorSubcoreMesh(core_axis_name="c", subcore_axis_name="s")

    @functools.partial(
        pl.kernel,
        mesh=mesh,
        out_type=jax.ShapeDtypeStruct((T, G), jnp.float32),
        scratch_types=[
            pltpu.VMEM((nchunk, chunk), jnp.int32),
            pltpu.VMEM((chunk, G), jnp.float32),
            pltpu.VMEM((chunk, G), jnp.float32),
            pltpu.VMEM((chunk, G), jnp.float32),
            pltpu.SemaphoreType.DMA,
            pltpu.SemaphoreType.DMA,
            pltpu.SemaphoreType.DMA,
            pltpu.SemaphoreType.DMA,
            pltpu.SemaphoreType.DMA,
            pltpu.SemaphoreType.DMA,
        ],
    )
    def gather_k(idx_hbm, table_hbm, out_hbm, idx_v, rows_a, rows_b, rows_c,
                 sg_a, sg_b, sg_c, sw_a, sw_b, sw_c):
        wid = lax.axis_index("s") * NC + lax.axis_index("c")
        base = wid * rows_per_w
        pltpu.sync_copy(idx_hbm.at[wid], idx_v)
        bufs = (rows_a, rows_b, rows_c)
        sg = (sg_a, sg_b, sg_c)
        sw = (sw_a, sw_b, sw_c)

        def gather(ci):
            return pltpu.async_copy(
                table_hbm.at[idx_v.at[ci]], bufs[ci % 3], sg[ci % 3])

        gcp = [gather(0), gather(1)]
        wcp = []
        for ci in range(nchunk):
            gcp[ci].wait()
            wcp.append(pltpu.async_copy(
                bufs[ci % 3], out_hbm.at[pl.ds(base + ci * chunk, chunk)],
                sw[ci % 3]))
            if ci + 2 < nchunk:
                if ci >= 1:
                    wcp[ci - 1].wait()
                gcp.append(gather(ci + 2))
        for j in range(nchunk - 3, nchunk):
            wcp[j].wait()

    return gather_k


def kernel(predicted_positions, real_positions, real_expressions):
    B, N, D = predicted_positions.shape
    G = real_expressions.shape[-1]

    predT = jnp.transpose(predicted_positions, (0, 2, 1))

    idx = _nearest_indices(real_positions, predT, bj=2048)

    table = real_expressions.reshape(B * N, G)
    chunk = 64
    nw = 32
    idx3 = idx.reshape(nw, (B * N) // (nw * chunk), chunk)
    gather_fn = _make_sc_gather(B * N, G, chunk=chunk)
    out = gather_fn(idx3, table)
    return out.reshape(B, N, G)


# speedup vs baseline: 1.0692x; 1.0030x over previous
import functools

import jax
import jax.numpy as jnp
from jax import lax
from jax.experimental import pallas as pl
from jax.experimental.pallas import tpu as pltpu
from jax.experimental.pallas import tpu_sc as plsc

def _nn_body(n_total, bj, nj, real_ref, predT_ref, idx_ref, vmin_ref, hi_ref,
             idxf_ref, rows_ref):
    b = pl.program_id(0)
    p = pl.program_id(1)
    jj = pl.program_id(2)

    @pl.when((b == 0) & (p == 0) & (jj == 0))
    def _init_rows():
        ri = lax.broadcasted_iota(jnp.int32, (bj, n_total), 0)
        rows_ref[...] = lax.bitcast_convert_type(ri | 0x3F800000, jnp.float32)

    def _sq():
        r = real_ref[0]
        pT = predT_ref[0]
        r_sq = (r[:, 0:1] * r[:, 0:1] + r[:, 1:2] * r[:, 1:2]
                + r[:, 2:3] * r[:, 2:3])
        p_sq = (pT[0:1, :] * pT[0:1, :] + pT[1:2, :] * pT[1:2, :]
                + pT[2:3, :] * pT[2:3, :])
        cross2 = lax.dot_general(
            r + r, pT, dimension_numbers=(((1,), (0,)), ((), ())),
            preferred_element_type=jnp.float32)
        return (p_sq + r_sq) - cross2

    @pl.when(p == 0)
    def _pass_min():
        m = jnp.min(_sq(), axis=0, keepdims=True)

        @pl.when(jj == 0)
        def _():
            vmin_ref[...] = m

        @pl.when(jj != 0)
        def _():
            vmin_ref[...] = jnp.minimum(vmin_ref[...], m)

    @pl.when(p == 1)
    def _pass_idx():
        @pl.when(jj == 0)
        def _():
            vmin = vmin_ref[...]
            dmin = jnp.sqrt(jnp.maximum(vmin, 0.0))
            c = lax.bitcast_convert_type(dmin * dmin, jnp.int32) + 3
            for _ in range(6):
                cf = lax.bitcast_convert_type(c, jnp.float32)
                c = c - (jnp.sqrt(cf) > dmin).astype(jnp.int32)
            hi = lax.bitcast_convert_type(c, jnp.float32)
            hi_ref[...] = jnp.where(dmin > 0.0, hi, 0.0)
            idxf_ref[...] = jnp.full((1, n_total), 1 << 30, jnp.int32)

        acc = _sq() <= hi_ref[...]
        enc = jnp.where(acc, rows_ref[...], jnp.float32(3.0))
        cand = jnp.min(enc, axis=0, keepdims=True)
        dec = lax.bitcast_convert_type(cand, jnp.int32) & 0x7FFFFF
        merged = jnp.minimum(idxf_ref[...], dec + (b * n_total + jj * bj))
        idxf_ref[...] = merged

        @pl.when(jj == nj - 1)
        def _():
            idx_ref[0] = merged


def _nearest_indices(real_pad, predT_pad, bj):
    B, N, _ = real_pad.shape
    nj = N // bj
    return pl.pallas_call(
        functools.partial(_nn_body, N, bj, nj),
        grid=(B, 2, nj),
        in_specs=[
            pl.BlockSpec((1, bj, 3), lambda b, p, j: (b, j, 0)),
            pl.BlockSpec((1, 3, N), lambda b, p, j: (b, 0, 0)),
        ],
        out_specs=pl.BlockSpec((1, 1, N), lambda b, p, j: (b, 0, 0)),
        out_shape=jax.ShapeDtypeStruct((B, 1, N), jnp.int32),
        scratch_shapes=[
            pltpu.VMEM((1, N), jnp.float32),
            pltpu.VMEM((1, N), jnp.float32),
            pltpu.VMEM((1, N), jnp.int32),
            pltpu.VMEM((bj, N), jnp.float32),
        ],
        compiler_params=pltpu.CompilerParams(
            dimension_semantics=("parallel", "arbitrary", "arbitrary")),
    )(real_pad, predT_pad)


def _make_sc_gather(T, G, chunk):
    NC, NS = 2, 16
    nw = NC * NS
    rows_per_w = T // nw
    nchunk = rows_per_w // chunk
    mesh = plsc.VectorSubcoreMesh(core_axis_name="c", subcore_axis_name="s")

    @functools.partial(
        pl.kernel,
        mesh=mesh,
        out_type=jax.ShapeDtypeStruct((T, G), jnp.float32),
        scratch_types=[
            pltpu.VMEM((nchunk, chunk), jnp.int32),
            pltpu.VMEM((chunk, G), jnp.float32),
            pltpu.VMEM((chunk, G), jnp.float32),
            pltpu.VMEM((chunk, G), jnp.float32),
            pltpu.SemaphoreType.DMA,
            pltpu.SemaphoreType.DMA,
            pltpu.SemaphoreType.DMA,
            pltpu.SemaphoreType.DMA,
            pltpu.SemaphoreType.DMA,
            pltpu.SemaphoreType.DMA,
        ],
    )
    def gather_k(idx_hbm, table_hbm, out_hbm, idx_v, rows_a, rows_b, rows_c,
                 sg_a, sg_b, sg_c, sw_a, sw_b, sw_c):
        wid = lax.axis_index("s") * NC + lax.axis_index("c")
        base = wid * rows_per_w
        pltpu.sync_copy(idx_hbm.at[wid], idx_v)
        bufs = (rows_a, rows_b, rows_c)
        sg = (sg_a, sg_b, sg_c)
        sw = (sw_a, sw_b, sw_c)

        def gather(ci):
            return pltpu.async_copy(
                table_hbm.at[idx_v.at[ci]], bufs[ci % 3], sg[ci % 3])

        gcp = [gather(0), gather(1)]
        wcp = []
        for ci in range(nchunk):
            gcp[ci].wait()
            wcp.append(pltpu.async_copy(
                bufs[ci % 3], out_hbm.at[pl.ds(base + ci * chunk, chunk)],
                sw[ci % 3]))
            if ci + 2 < nchunk:
                if ci >= 1:
                    wcp[ci - 1].wait()
                gcp.append(gather(ci + 2))
        for j in range(nchunk - 3, nchunk):
            wcp[j].wait()

    return gather_k


def kernel(predicted_positions, real_positions, real_expressions):
    B, N, D = predicted_positions.shape
    G = real_expressions.shape[-1]

    predT = jnp.transpose(predicted_positions, (0, 2, 1))

    idx = _nearest_indices(real_positions, predT, bj=2048)

    table = real_expressions.reshape(B * N, G)
    chunk = 64
    nw = 32
    idx3 = idx.reshape(nw, (B * N) // (nw * chunk), chunk)
    gather_fn = _make_sc_gather(B * N, G, chunk=chunk)
    out = gather_fn(idx3, table)
    return out.reshape(B, N, G)
